# baseline (speedup 1.0000x reference)
import jax
import jax.numpy as jnp
from jax.experimental import pallas as pl
from jax.experimental.pallas import tpu as pltpu


def _gcn_body(adj_a_ref, adj_b_ref, x_ref, w_ref, b_ref, out_ref):
    wt = w_ref[...].T
    b = b_ref[...]
    agg_a = jnp.dot(adj_a_ref[...], x_ref[...], preferred_element_type=jnp.float32)
    out_ref[0] = jnp.dot(agg_a, wt, preferred_element_type=jnp.float32) + b
    agg_b = jnp.dot(adj_b_ref[...], x_ref[...], preferred_element_type=jnp.float32)
    out_ref[1] = jnp.dot(agg_b, wt, preferred_element_type=jnp.float32) + b


def kernel(x, adj, W, bias):
    n, d_in = x.shape
    d_out = W.shape[0]
    bm = 200
    half = n // (2 * bm)

    out = pl.pallas_call(
        _gcn_body,
        grid=(half,),
        in_specs=[
            pl.BlockSpec((bm, n), lambda i: (i, 0)),
            pl.BlockSpec((bm, n), lambda i: (i + half, 0)),
            pl.BlockSpec((n, d_in), lambda i: (0, 0)),
            pl.BlockSpec((d_out, d_in), lambda i: (0, 0)),
            pl.BlockSpec((1, d_out), lambda i: (0, 0)),
        ],
        out_specs=pl.BlockSpec((2, bm, d_out), lambda i: (0, i, 0)),
        out_shape=jax.ShapeDtypeStruct((2, n // 2, d_out), jnp.float32),
        compiler_params=pltpu.CompilerParams(
            vmem_limit_bytes=60 * 1024 * 1024,
        ),
    )(adj, adj, x, W, bias.reshape(1, d_out))
    return out.reshape(n, d_out)


# speedup vs baseline: 1.0951x; 1.0951x over previous
import jax
import jax.numpy as jnp
from jax.experimental import pallas as pl
from jax.experimental.pallas import tpu as pltpu


def _gcn_body(adj_ref, x_ref, w_ref, b_ref, out_ref):
    agg = jnp.dot(adj_ref[...], x_ref[...], preferred_element_type=jnp.float32)
    out_ref[...] = (
        jnp.dot(agg, w_ref[...].T, preferred_element_type=jnp.float32)
        + b_ref[...]
    )


def kernel(x, adj, W, bias):
    n, d_in = x.shape
    d_out = W.shape[0]
    bm = 400

    out = pl.pallas_call(
        _gcn_body,
        grid=(n // bm,),
        in_specs=[
            pl.BlockSpec((bm, n), lambda i: (i, 0)),
            pl.BlockSpec((n, d_in), lambda i: (0, 0)),
            pl.BlockSpec((d_out, d_in), lambda i: (0, 0)),
            pl.BlockSpec((1, d_out), lambda i: (0, 0)),
        ],
        out_specs=pl.BlockSpec((bm, d_out), lambda i: (i, 0)),
        out_shape=jax.ShapeDtypeStruct((n, d_out), jnp.float32),
        compiler_params=pltpu.CompilerParams(
            vmem_limit_bytes=60 * 1024 * 1024,
        ),
    )(adj, x, W, bias.reshape(1, d_out))
    return out
